# baseline (speedup 1.0000x reference)
import functools

import jax
import jax.numpy as jnp
from jax import lax
from jax.experimental import pallas as pl
from jax.experimental.pallas import tpu as pltpu
from jax.experimental.pallas import tpu_sc as plsc

_NC = 2
_NS = 16
_NW = _NC * _NS

_BATCH = 4
_MAXLEN = 2048
_EMBED = 1024

_ROWS_PER_W = _MAXLEN // _NW
_TILE_ROWS = 32
_TILE_ELEMS = _TILE_ROWS * _EMBED
_P_TILES = _ROWS_PER_W // _TILE_ROWS


def _sc_add_kernel(x_hbm, pos_hbm, out_hbm, pos_v, x_v, sem):
    wid = lax.axis_index("c") * _NS + lax.axis_index("s")
    pos_base = wid * _ROWS_PER_W * _EMBED
    for p in range(_P_TILES):
        pltpu.sync_copy(
            pos_hbm.at[pl.ds(pos_base + p * _TILE_ELEMS, _TILE_ELEMS)], pos_v
        )
        for b in range(_BATCH):
            x_base = b * (_MAXLEN * _EMBED) + pos_base + p * _TILE_ELEMS
            pltpu.sync_copy(x_hbm.at[pl.ds(x_base, _TILE_ELEMS)], x_v)

            def _body(i, _):
                sl = pl.ds(i * 16, 16)
                x_v[sl] = x_v[sl] + pos_v[sl]
                return 0

            lax.fori_loop(0, _TILE_ELEMS // 16, _body, 0)
            pltpu.sync_copy(x_v, out_hbm.at[pl.ds(x_base, _TILE_ELEMS)])


def kernel(x, pos_table):
    batch, maxlen, embed = x.shape
    xf = x.reshape(batch * maxlen * embed)
    pf = pos_table.reshape(maxlen * embed)
    mesh = plsc.VectorSubcoreMesh(core_axis_name="c", subcore_axis_name="s")
    k = functools.partial(
        pl.kernel,
        mesh=mesh,
        out_type=jax.ShapeDtypeStruct(xf.shape, xf.dtype),
        scratch_types=[
            pltpu.VMEM((_TILE_ELEMS,), jnp.float32),
            pltpu.VMEM((_TILE_ELEMS,), jnp.float32),
            pltpu.SemaphoreType.DMA,
        ],
    )(_sc_add_kernel)
    out = k(xf, pf)
    return out.reshape(x.shape)


# speedup vs baseline: 1.6564x; 1.6564x over previous
import functools

import jax
import jax.numpy as jnp
from jax import lax
from jax.experimental import pallas as pl
from jax.experimental.pallas import tpu as pltpu
from jax.experimental.pallas import tpu_sc as plsc

_NC = 2
_NS = 16
_NW = _NC * _NS

_MAXLEN = 2048
_EMBED = 1024

_ROWS_PER_W = _MAXLEN // _NW
_TILE_ROWS = 32
_TILE_ELEMS = _TILE_ROWS * _EMBED
_P_TILES = _ROWS_PER_W // _TILE_ROWS

_SEQ_BLK = 512


def _tc_add_kernel(x_ref, pos_ref, o_ref):
    o_ref[...] = x_ref[...] + pos_ref[...][None, :, :]


def _sc_add_kernel(x_hbm, pos_hbm, out_hbm, pos_v, x_v, sem):
    wid = lax.axis_index("c") * _NS + lax.axis_index("s")
    base = wid * _ROWS_PER_W * _EMBED
    for p in range(_P_TILES):
        off = base + p * _TILE_ELEMS
        pltpu.sync_copy(pos_hbm.at[pl.ds(off, _TILE_ELEMS)], pos_v)
        pltpu.sync_copy(x_hbm.at[pl.ds(off, _TILE_ELEMS)], x_v)

        def _body(i, _):
            sl = pl.ds(i * 16, 16)
            x_v[sl] = x_v[sl] + pos_v[sl]
            return 0

        lax.fori_loop(0, _TILE_ELEMS // 16, _body, 0)
        pltpu.sync_copy(x_v, out_hbm.at[pl.ds(off, _TILE_ELEMS)])


def kernel(x, pos_table):
    batch, maxlen, embed = x.shape
    x_tc = x[: batch - 1]
    x_sc = x[batch - 1].reshape(maxlen * embed)
    pf = pos_table.reshape(maxlen * embed)

    mesh = plsc.VectorSubcoreMesh(core_axis_name="c", subcore_axis_name="s")
    sc_k = functools.partial(
        pl.kernel,
        mesh=mesh,
        out_type=jax.ShapeDtypeStruct(x_sc.shape, x_sc.dtype),
        scratch_types=[
            pltpu.VMEM((_TILE_ELEMS,), jnp.float32),
            pltpu.VMEM((_TILE_ELEMS,), jnp.float32),
            pltpu.SemaphoreType.DMA,
        ],
    )(_sc_add_kernel)
    out_sc = sc_k(x_sc, pf)

    out_tc = pl.pallas_call(
        _tc_add_kernel,
        grid=(maxlen // _SEQ_BLK,),
        in_specs=[
            pl.BlockSpec((batch - 1, _SEQ_BLK, embed), lambda i: (0, i, 0)),
            pl.BlockSpec((_SEQ_BLK, embed), lambda i: (i, 0)),
        ],
        out_specs=pl.BlockSpec((batch - 1, _SEQ_BLK, embed), lambda i: (0, i, 0)),
        out_shape=jax.ShapeDtypeStruct(x_tc.shape, x_tc.dtype),
        compiler_params=pltpu.CompilerParams(
            dimension_semantics=("parallel",),
        ),
    )(x_tc, pos_table)

    return jnp.concatenate(
        [out_tc, out_sc.reshape(1, maxlen, embed)], axis=0
    )


# speedup vs baseline: 8.4358x; 5.0930x over previous
import jax
import jax.numpy as jnp
from jax.experimental import pallas as pl
from jax.experimental.pallas import tpu as pltpu


def _add_kernel(x_ref, pos_ref, o_ref):
    o_ref[...] = x_ref[...] + pos_ref[...][None, :, :]


def kernel(x, pos_table):
    batch, maxlen, embed = x.shape
    return pl.pallas_call(
        _add_kernel,
        grid=(batch,),
        in_specs=[
            pl.BlockSpec((1, maxlen, embed), lambda i: (i, 0, 0)),
            pl.BlockSpec((maxlen, embed), lambda i: (0, 0)),
        ],
        out_specs=pl.BlockSpec((1, maxlen, embed), lambda i: (i, 0, 0)),
        out_shape=jax.ShapeDtypeStruct(x.shape, x.dtype),
        compiler_params=pltpu.CompilerParams(
            dimension_semantics=("arbitrary",),
        ),
    )(x, pos_table)
